# Initial kernel scaffold; baseline (speedup 1.0000x reference)
import jax
import jax.numpy as jnp
from jax.experimental import pallas as pl


def kernel(x, edge_index, W0, W1, b):
    raise NotImplementedError("write your pallas kernel here")


# speedup vs baseline: 269.5222x; 269.5222x over previous
import functools

import jax
import jax.numpy as jnp
from jax import lax
from jax.experimental import pallas as pl
from jax.experimental.pallas import tpu as pltpu
from jax.experimental.pallas import tpu_sc as plsc

N_NODES = 10000
N_EDGES = 640000
N_PAD = 10240
NC = 2
NS = 16
NW = NC * NS
CHUNK = N_EDGES // NW
L = 16

_mesh = plsc.VectorSubcoreMesh(core_axis_name="c", subcore_axis_name="s")
_sc_params = pltpu.CompilerParams(needs_layout_passes=False)


@functools.partial(
    pl.kernel,
    out_type=jax.ShapeDtypeStruct((2, NW, N_PAD), jnp.float32),
    mesh=_mesh,
    scratch_types=[
        pltpu.VMEM((CHUNK,), jnp.int32),
        pltpu.VMEM((CHUNK,), jnp.int32),
        pltpu.VMEM((N_PAD,), jnp.float32),
        pltpu.VMEM((N_PAD,), jnp.float32),
    ],
    compiler_params=_sc_params,
)
def _sc_histograms(row_hbm, col_hbm, out_hbm, row_v, col_v, deg_v, sl_v):
    wid = lax.axis_index("s") * NC + lax.axis_index("c")
    base = wid * CHUNK
    pltpu.sync_copy(row_hbm.at[pl.ds(base, CHUNK)], row_v)
    pltpu.sync_copy(col_hbm.at[pl.ds(base, CHUNK)], col_v)

    zeros = jnp.zeros((L,), jnp.float32)

    def zero_body(i, carry):
        deg_v[pl.ds(i * L, L)] = zeros
        sl_v[pl.ds(i * L, L)] = zeros
        return carry

    lax.fori_loop(0, N_PAD // L, zero_body, 0)

    ones = jnp.ones((L,), jnp.float32)

    def body(i, carry):
        r = row_v[pl.ds(i * L, L)]
        c = col_v[pl.ds(i * L, L)]
        plsc.addupdate_scatter(deg_v, [r], ones)
        plsc.addupdate_scatter(sl_v, [r], ones, mask=r == c)
        return carry

    lax.fori_loop(0, CHUNK // L, body, 0)

    pltpu.sync_copy(deg_v, out_hbm.at[0, wid])
    pltpu.sync_copy(sl_v, out_hbm.at[1, wid])


@functools.partial(
    pl.kernel,
    out_type=jax.ShapeDtypeStruct((NW, N_PAD), jnp.float32),
    mesh=_mesh,
    scratch_types=[
        pltpu.VMEM((CHUNK,), jnp.int32),
        pltpu.VMEM((CHUNK,), jnp.int32),
        pltpu.VMEM((N_PAD,), jnp.float32),
        pltpu.VMEM((N_PAD,), jnp.float32),
    ],
    compiler_params=_sc_params,
)
def _sc_weighted_hist(row_hbm, col_hbm, dis_hbm, out_hbm, row_v, col_v, dis_v, s_v):
    wid = lax.axis_index("s") * NC + lax.axis_index("c")
    base = wid * CHUNK
    pltpu.sync_copy(row_hbm.at[pl.ds(base, CHUNK)], row_v)
    pltpu.sync_copy(col_hbm.at[pl.ds(base, CHUNK)], col_v)
    pltpu.sync_copy(dis_hbm.at[0], dis_v)

    zeros = jnp.zeros((L,), jnp.float32)

    def zero_body(i, carry):
        s_v[pl.ds(i * L, L)] = zeros
        return carry

    lax.fori_loop(0, N_PAD // L, zero_body, 0)

    def body(i, carry):
        r = row_v[pl.ds(i * L, L)]
        c = col_v[pl.ds(i * L, L)]
        g = plsc.load_gather(dis_v, [c])
        plsc.addupdate_scatter(s_v, [r], g)
        return carry

    lax.fori_loop(0, CHUNK // L, body, 0)

    pltpu.sync_copy(s_v, out_hbm.at[wid])


def _tc_rsqrt_body(p_ref, out_ref):
    p = p_ref[...]
    deg = jnp.sum(p[:NW], axis=0, keepdims=True)
    sl = jnp.sum(p[NW:], axis=0, keepdims=True)
    dis = jnp.where(deg > 0, lax.rsqrt(jnp.maximum(deg, 1e-12)), 0.0)
    out_ref[...] = jnp.concatenate([dis, sl], axis=0)


def _tc_final_body(x_ref, sp_ref, dsl_ref, w0_ref, w1_ref, b_ref, out_ref):
    s = jnp.sum(sp_ref[...], axis=0, keepdims=True)
    dis = dsl_ref[0:1, :]
    sl = dsl_ref[1:2, :]
    c = -(dis * s) - sl
    c = c[:, :N_NODES]
    x = x_ref[...]
    v1 = jnp.dot(c, x, preferred_element_type=jnp.float32)
    v0 = jnp.sum(x, axis=0, keepdims=True)
    out = (
        jnp.dot(v0, w0_ref[...], preferred_element_type=jnp.float32)
        + jnp.dot(v1, w1_ref[...], preferred_element_type=jnp.float32)
    ) * (1.0 / N_NODES) + b_ref[...]
    out_ref[...] = out


@jax.jit
def kernel(x, edge_index, W0, W1, b):
    out_dim = W0.shape[1]

    row = edge_index[0]
    col = edge_index[1]

    partials = _sc_histograms(row, col)
    partials2d = partials.reshape(2 * NW, N_PAD)

    dsl = pl.pallas_call(
        _tc_rsqrt_body,
        out_shape=jax.ShapeDtypeStruct((2, N_PAD), jnp.float32),
    )(partials2d)

    s_partials = _sc_weighted_hist(row, col, dsl)

    out = pl.pallas_call(
        _tc_final_body,
        out_shape=jax.ShapeDtypeStruct((1, out_dim), jnp.float32),
    )(x, s_partials, dsl, W0, W1, b.reshape(1, out_dim))
    return out


# speedup vs baseline: 353.1876x; 1.3104x over previous
import functools

import jax
import jax.numpy as jnp
from jax import lax
from jax.experimental import pallas as pl
from jax.experimental.pallas import tpu as pltpu
from jax.experimental.pallas import tpu_sc as plsc

N_NODES = 10000
N_EDGES = 640000
N_PAD = 10240
NC = 2
NS = 16
NW = NC * NS
CHUNK = N_EDGES // NW
L = 16

_mesh = plsc.VectorSubcoreMesh(core_axis_name="c", subcore_axis_name="s")
_sc_params = pltpu.CompilerParams(needs_layout_passes=False)


@functools.partial(
    pl.kernel,
    out_type=jax.ShapeDtypeStruct((2, NW, N_PAD), jnp.float32),
    mesh=_mesh,
    scratch_types=[
        pltpu.VMEM((CHUNK,), jnp.int32),
        pltpu.VMEM((CHUNK,), jnp.int32),
        pltpu.VMEM((N_PAD,), jnp.float32),
        pltpu.VMEM((N_PAD,), jnp.float32),
    ],
    compiler_params=_sc_params,
)
def _sc_histograms(row_hbm, col_hbm, out_hbm, row_v, col_v, deg_v, sl_v):
    wid = lax.axis_index("s") * NC + lax.axis_index("c")
    base = wid * CHUNK
    pltpu.sync_copy(row_hbm.at[pl.ds(base, CHUNK)], row_v)
    pltpu.sync_copy(col_hbm.at[pl.ds(base, CHUNK)], col_v)

    zeros = jnp.zeros((L,), jnp.float32)

    def zero_body(i, carry):
        for u in range(8):
            deg_v[pl.ds((i * 8 + u) * L, L)] = zeros
            sl_v[pl.ds((i * 8 + u) * L, L)] = zeros
        return carry

    lax.fori_loop(0, N_PAD // L // 8, zero_body, 0)

    ones = jnp.ones((L,), jnp.float32)
    UNROLL = 8

    def body(i, carry):
        rs = []
        cs = []
        for u in range(UNROLL):
            rs.append(row_v[pl.ds((i * UNROLL + u) * L, L)])
            cs.append(col_v[pl.ds((i * UNROLL + u) * L, L)])
        for u in range(UNROLL):
            plsc.addupdate_scatter(deg_v, [rs[u]], ones)
        for u in range(UNROLL):
            plsc.addupdate_scatter(sl_v, [rs[u]], ones, mask=rs[u] == cs[u])
        return carry

    lax.fori_loop(0, CHUNK // L // UNROLL, body, 0)

    pltpu.sync_copy(deg_v, out_hbm.at[0, wid])
    pltpu.sync_copy(sl_v, out_hbm.at[1, wid])


@functools.partial(
    pl.kernel,
    out_type=jax.ShapeDtypeStruct((NW, N_PAD), jnp.float32),
    mesh=_mesh,
    scratch_types=[
        pltpu.VMEM((CHUNK,), jnp.int32),
        pltpu.VMEM((CHUNK,), jnp.int32),
        pltpu.VMEM((N_PAD,), jnp.float32),
        pltpu.VMEM((N_PAD,), jnp.float32),
    ],
    compiler_params=_sc_params,
)
def _sc_weighted_hist(row_hbm, col_hbm, dis_hbm, out_hbm, row_v, col_v, dis_v, s_v):
    wid = lax.axis_index("s") * NC + lax.axis_index("c")
    base = wid * CHUNK
    pltpu.sync_copy(row_hbm.at[pl.ds(base, CHUNK)], row_v)
    pltpu.sync_copy(col_hbm.at[pl.ds(base, CHUNK)], col_v)
    pltpu.sync_copy(dis_hbm.at[0], dis_v)

    zeros = jnp.zeros((L,), jnp.float32)

    def zero_body(i, carry):
        for u in range(8):
            s_v[pl.ds((i * 8 + u) * L, L)] = zeros
        return carry

    lax.fori_loop(0, N_PAD // L // 8, zero_body, 0)

    UNROLL = 8

    def body(i, carry):
        rs = []
        gs = []
        for u in range(UNROLL):
            rs.append(row_v[pl.ds((i * UNROLL + u) * L, L)])
            cu = col_v[pl.ds((i * UNROLL + u) * L, L)]
            gs.append(plsc.load_gather(dis_v, [cu]))
        for u in range(UNROLL):
            plsc.addupdate_scatter(s_v, [rs[u]], gs[u])
        return carry

    lax.fori_loop(0, CHUNK // L // UNROLL, body, 0)

    pltpu.sync_copy(s_v, out_hbm.at[wid])


def _tc_rsqrt_body(p_ref, out_ref):
    p = p_ref[...]
    deg = jnp.sum(p[:NW], axis=0, keepdims=True)
    sl = jnp.sum(p[NW:], axis=0, keepdims=True)
    dis = jnp.where(deg > 0, lax.rsqrt(jnp.maximum(deg, 1e-12)), 0.0)
    out_ref[...] = jnp.concatenate([dis, sl], axis=0)


def _tc_final_body(x_ref, sp_ref, dsl_ref, w0_ref, w1_ref, b_ref, out_ref):
    s = jnp.sum(sp_ref[...], axis=0, keepdims=True)
    dis = dsl_ref[0:1, :]
    sl = dsl_ref[1:2, :]
    c = -(dis * s) - sl
    c = c[:, :N_NODES]
    x = x_ref[...]
    v1 = jnp.dot(c, x, preferred_element_type=jnp.float32)
    v0 = jnp.sum(x, axis=0, keepdims=True)
    out = (
        jnp.dot(v0, w0_ref[...], preferred_element_type=jnp.float32)
        + jnp.dot(v1, w1_ref[...], preferred_element_type=jnp.float32)
    ) * (1.0 / N_NODES) + b_ref[...]
    out_ref[...] = out


@jax.jit
def kernel(x, edge_index, W0, W1, b):
    out_dim = W0.shape[1]

    row = edge_index[0]
    col = edge_index[1]

    partials = _sc_histograms(row, col)
    partials2d = partials.reshape(2 * NW, N_PAD)

    dsl = pl.pallas_call(
        _tc_rsqrt_body,
        out_shape=jax.ShapeDtypeStruct((2, N_PAD), jnp.float32),
    )(partials2d)

    s_partials = _sc_weighted_hist(row, col, dsl)

    out = pl.pallas_call(
        _tc_final_body,
        out_shape=jax.ShapeDtypeStruct((1, out_dim), jnp.float32),
    )(x, s_partials, dsl, W0, W1, b.reshape(1, out_dim))
    return out


# speedup vs baseline: 360.9346x; 1.0219x over previous
import functools

import jax
import jax.numpy as jnp
from jax import lax
from jax.experimental import pallas as pl
from jax.experimental.pallas import tpu as pltpu
from jax.experimental.pallas import tpu_sc as plsc

N_NODES = 10000
N_EDGES = 640000
N_PAD = 10240
NC = 2
NS = 16
NW = NC * NS
CHUNK = N_EDGES // NW
L = 16
SLICE = N_PAD // NS
UNROLL = 8

_mesh = plsc.VectorSubcoreMesh(core_axis_name="c", subcore_axis_name="s")
_sc_params = pltpu.CompilerParams(needs_layout_passes=False)


@functools.partial(
    pl.kernel,
    out_type=(
        jax.ShapeDtypeStruct((NW, N_PAD), jnp.float32),
        jax.ShapeDtypeStruct((NW, N_PAD), jnp.float32),
    ),
    mesh=_mesh,
    scratch_types=[
        pltpu.VMEM((CHUNK,), jnp.int32),
        pltpu.VMEM((CHUNK,), jnp.int32),
        pltpu.VMEM((N_PAD,), jnp.float32),
        pltpu.VMEM((N_PAD,), jnp.float32),
        pltpu.SemaphoreType.DMA,
    ],
    compiler_params=_sc_params,
)
def _sc_histograms(row_hbm, col_hbm, deg_hbm, sl_hbm, row_v, col_v, deg_v,
                   sl_v, sem):
    wid = lax.axis_index("s") * NC + lax.axis_index("c")
    base = wid * CHUNK
    cp_r = pltpu.async_copy(row_hbm.at[pl.ds(base, CHUNK)], row_v, sem)
    cp_c = pltpu.async_copy(col_hbm.at[pl.ds(base, CHUNK)], col_v, sem)

    zeros = jnp.zeros((L,), jnp.float32)

    def zero_body(i, carry):
        for u in range(8):
            deg_v[pl.ds((i * 8 + u) * L, L)] = zeros
            sl_v[pl.ds((i * 8 + u) * L, L)] = zeros
        return carry

    lax.fori_loop(0, N_PAD // L // 8, zero_body, 0)
    cp_r.wait()
    cp_c.wait()

    ones = jnp.ones((L,), jnp.float32)

    def body(i, carry):
        rs = []
        cs = []
        for u in range(UNROLL):
            rs.append(row_v[pl.ds((i * UNROLL + u) * L, L)])
            cs.append(col_v[pl.ds((i * UNROLL + u) * L, L)])
        for u in range(UNROLL):
            plsc.addupdate_scatter(deg_v, [rs[u]], ones)
        for u in range(UNROLL):
            plsc.addupdate_scatter(sl_v, [rs[u]], ones, mask=rs[u] == cs[u])
        return carry

    lax.fori_loop(0, CHUNK // L // UNROLL, body, 0)

    pltpu.sync_copy(deg_v, deg_hbm.at[wid])
    pltpu.sync_copy(sl_v, sl_hbm.at[wid])


def _newton_rsqrt(x):
    i = plsc.bitcast(x, jnp.int32)
    i = 0x5F3759DF - lax.shift_right_logical(i, 1)
    y = plsc.bitcast(i, jnp.float32)
    half = x * -0.5
    for _ in range(3):
        y = y * (y * y * half + 1.5)
    return y


@functools.partial(
    pl.kernel,
    out_type=(
        jax.ShapeDtypeStruct((NW, N_PAD), jnp.float32),
        jax.ShapeDtypeStruct((N_PAD,), jnp.float32),
    ),
    mesh=_mesh,
    scratch_types=[
        pltpu.VMEM((CHUNK,), jnp.int32),
        pltpu.VMEM((CHUNK,), jnp.int32),
        pltpu.VMEM((NW, SLICE), jnp.float32),
        pltpu.VMEM((N_PAD,), jnp.float32),
        pltpu.VMEM((N_PAD,), jnp.float32),
        pltpu.VMEM_SHARED((N_PAD,), jnp.float32),
        pltpu.SemaphoreType.DMA,
        pltpu.SemaphoreType.DMA,
    ],
    compiler_params=_sc_params,
)
def _sc_weighted_hist(row_hbm, col_hbm, degp_hbm, out_hbm, dis_hbm, row_v,
                      col_v, part_v, dis_v, s_v, shared, sem, sem2):
    cid = lax.axis_index("c")
    sid = lax.axis_index("s")
    wid = sid * NC + cid
    base = wid * CHUNK
    cp_r = pltpu.async_copy(row_hbm.at[pl.ds(base, CHUNK)], row_v, sem)
    cp_c = pltpu.async_copy(col_hbm.at[pl.ds(base, CHUNK)], col_v, sem)

    col0 = sid * SLICE
    pltpu.sync_copy(degp_hbm.at[:, pl.ds(col0, SLICE)], part_v)

    def red_body(j, carry):
        acc = part_v[0, pl.ds(j * L, L)]
        for t in range(1, NW):
            acc = acc + part_v[t, pl.ds(j * L, L)]
        dis = _newton_rsqrt(acc)
        dis = jnp.where(acc > 0.0, dis, 0.0)
        dis_v[pl.ds(col0 + j * L, L)] = dis
        return carry

    lax.fori_loop(0, SLICE // L, red_body, 0)

    pltpu.sync_copy(dis_v.at[pl.ds(col0, SLICE)], shared.at[pl.ds(col0, SLICE)])

    @pl.when(cid == 0)
    def _():
        pltpu.sync_copy(dis_v.at[pl.ds(col0, SLICE)],
                        dis_hbm.at[pl.ds(col0, SLICE)])

    plsc.subcore_barrier()
    pltpu.sync_copy(shared, dis_v)

    zeros = jnp.zeros((L,), jnp.float32)

    def zero_body(i, carry):
        for u in range(8):
            s_v[pl.ds((i * 8 + u) * L, L)] = zeros
        return carry

    lax.fori_loop(0, N_PAD // L // 8, zero_body, 0)
    cp_r.wait()
    cp_c.wait()

    def body(i, carry):
        rs = []
        gs = []
        for u in range(UNROLL):
            rs.append(row_v[pl.ds((i * UNROLL + u) * L, L)])
            cu = col_v[pl.ds((i * UNROLL + u) * L, L)]
            gs.append(plsc.load_gather(dis_v, [cu]))
        for u in range(UNROLL):
            plsc.addupdate_scatter(s_v, [rs[u]], gs[u])
        return carry

    lax.fori_loop(0, CHUNK // L // UNROLL, body, 0)

    pltpu.sync_copy(s_v, out_hbm.at[wid])


def _tc_final_body(x_ref, sp_ref, slp_ref, dis_ref, w0_ref, w1_ref, b_ref,
                   out_ref):
    s = jnp.sum(sp_ref[...], axis=0, keepdims=True)
    sl = jnp.sum(slp_ref[...], axis=0, keepdims=True)
    dis = dis_ref[...]
    c = -(dis * s) - sl
    c = c[:, :N_NODES]
    x = x_ref[...]
    v1 = jnp.dot(c, x, preferred_element_type=jnp.float32)
    v0 = jnp.sum(x, axis=0, keepdims=True)
    out = (
        jnp.dot(v0, w0_ref[...], preferred_element_type=jnp.float32)
        + jnp.dot(v1, w1_ref[...], preferred_element_type=jnp.float32)
    ) * (1.0 / N_NODES) + b_ref[...]
    out_ref[...] = out


@jax.jit
def kernel(x, edge_index, W0, W1, b):
    out_dim = W0.shape[1]

    row = edge_index[0]
    col = edge_index[1]

    deg_p, sl_p = _sc_histograms(row, col)
    s_p, dis = _sc_weighted_hist(row, col, deg_p)

    out = pl.pallas_call(
        _tc_final_body,
        out_shape=jax.ShapeDtypeStruct((1, out_dim), jnp.float32),
    )(x, s_p, sl_p, dis.reshape(1, N_PAD), W0, W1, b.reshape(1, out_dim))
    return out


# speedup vs baseline: 395.1853x; 1.0949x over previous
import functools

import jax
import jax.numpy as jnp
from jax import lax
from jax.experimental import pallas as pl
from jax.experimental.pallas import tpu as pltpu
from jax.experimental.pallas import tpu_sc as plsc

N_NODES = 10000
N_EDGES = 640000
N_PAD = 10240
NC = 2
NS = 16
NW = NC * NS
CHUNK = N_EDGES // NW
DCHUNK = N_EDGES // NS
L = 16
SLICE = N_PAD // NS
UNROLL_A = 10
UNROLL_C = 10

_mesh = plsc.VectorSubcoreMesh(core_axis_name="c", subcore_axis_name="s")
_sc_params = pltpu.CompilerParams(needs_layout_passes=False)


def _newton_rsqrt(x):
    i = plsc.bitcast(x, jnp.int32)
    i = 0x5F3759DF - lax.shift_right_logical(i, 1)
    y = plsc.bitcast(i, jnp.float32)
    half = x * -0.5
    for _ in range(3):
        y = y * (y * y * half + 1.5)
    return y


@functools.partial(
    pl.kernel,
    out_type=(
        jax.ShapeDtypeStruct((NW, N_PAD), jnp.float32),
        jax.ShapeDtypeStruct((NW, N_PAD), jnp.float32),
        jax.ShapeDtypeStruct((N_PAD,), jnp.float32),
    ),
    mesh=_mesh,
    scratch_types=[
        pltpu.VMEM((DCHUNK,), jnp.int32),
        pltpu.VMEM((CHUNK,), jnp.int32),
        pltpu.VMEM((N_PAD,), jnp.float32),
        pltpu.VMEM((N_PAD,), jnp.float32),
        pltpu.VMEM((N_PAD,), jnp.float32),
        pltpu.VMEM((NS, SLICE), jnp.float32),
        pltpu.VMEM_SHARED((NS, N_PAD), jnp.float32),
        pltpu.VMEM_SHARED((N_PAD,), jnp.float32),
        pltpu.SemaphoreType.DMA,
        pltpu.SemaphoreType.DMA,
    ],
    compiler_params=_sc_params,
)
def _sc_edges(row_hbm, col_hbm, s_hbm, sl_hbm, dis_hbm, row_v, col_v, deg_v,
              s_v, sl_v, part_v, degsh, dissh, sem, sem2):
    cid = lax.axis_index("c")
    sid = lax.axis_index("s")
    wid = sid * NC + cid
    cp_r = pltpu.async_copy(row_hbm.at[pl.ds(sid * DCHUNK, DCHUNK)], row_v, sem)
    cp_c = pltpu.async_copy(col_hbm.at[pl.ds(wid * CHUNK, CHUNK)], col_v, sem2)

    zeros = jnp.zeros((L,), jnp.float32)

    def zero_body(i, carry):
        for u in range(8):
            deg_v[pl.ds((i * 8 + u) * L, L)] = zeros
            s_v[pl.ds((i * 8 + u) * L, L)] = zeros
            sl_v[pl.ds((i * 8 + u) * L, L)] = zeros
        return carry

    lax.fori_loop(0, N_PAD // L // 8, zero_body, 0)
    cp_r.wait()

    ones = jnp.ones((L,), jnp.float32)

    def deg_body(i, carry):
        rs = [row_v[pl.ds((i * UNROLL_A + u) * L, L)] for u in range(UNROLL_A)]
        for u in range(UNROLL_A):
            plsc.addupdate_scatter(deg_v, [rs[u]], ones)
        return carry

    lax.fori_loop(0, DCHUNK // L // UNROLL_A, deg_body, 0)

    pltpu.sync_copy(deg_v, degsh.at[sid])
    plsc.subcore_barrier()
    col0 = sid * SLICE
    pltpu.sync_copy(degsh.at[:, pl.ds(col0, SLICE)], part_v)

    def red_body(j, carry):
        acc = part_v[0, pl.ds(j * L, L)]
        for t in range(1, NS):
            acc = acc + part_v[t, pl.ds(j * L, L)]
        dis = _newton_rsqrt(acc)
        dis = jnp.where(acc > 0.0, dis, 0.0)
        deg_v[pl.ds(col0 + j * L, L)] = dis
        return carry

    lax.fori_loop(0, SLICE // L, red_body, 0)

    pltpu.sync_copy(deg_v.at[pl.ds(col0, SLICE)], dissh.at[pl.ds(col0, SLICE)])

    @pl.when(cid == 0)
    def _():
        pltpu.sync_copy(deg_v.at[pl.ds(col0, SLICE)],
                        dis_hbm.at[pl.ds(col0, SLICE)])

    plsc.subcore_barrier()
    pltpu.sync_copy(dissh, deg_v)
    cp_c.wait()

    cbase = cid * CHUNK

    def s_body(i, carry):
        rs = []
        cs = []
        gs = []
        for u in range(UNROLL_C):
            off = (i * UNROLL_C + u) * L
            rs.append(row_v[pl.ds(cbase + off, L)])
            cs.append(col_v[pl.ds(off, L)])
            gs.append(plsc.load_gather(deg_v, [cs[u]]))
        for u in range(UNROLL_C):
            plsc.addupdate_scatter(s_v, [rs[u]], gs[u])
            plsc.addupdate_scatter(sl_v, [rs[u]], ones, mask=rs[u] == cs[u])
        return carry

    lax.fori_loop(0, CHUNK // L // UNROLL_C, s_body, 0)

    pltpu.sync_copy(s_v, s_hbm.at[wid])
    pltpu.sync_copy(sl_v, sl_hbm.at[wid])


def _tc_final_body(x_ref, sp_ref, slp_ref, dis_ref, w0_ref, w1_ref, b_ref,
                   out_ref):
    s = jnp.sum(sp_ref[...], axis=0, keepdims=True)
    sl = jnp.sum(slp_ref[...], axis=0, keepdims=True)
    dis = dis_ref[...]
    c = -(dis * s) - sl
    c = c[:, :N_NODES]
    x = x_ref[...]
    v1 = jnp.dot(c, x, preferred_element_type=jnp.float32)
    v0 = jnp.sum(x, axis=0, keepdims=True)
    out = (
        jnp.dot(v0, w0_ref[...], preferred_element_type=jnp.float32)
        + jnp.dot(v1, w1_ref[...], preferred_element_type=jnp.float32)
    ) * (1.0 / N_NODES) + b_ref[...]
    out_ref[...] = out


@jax.jit
def kernel(x, edge_index, W0, W1, b):
    out_dim = W0.shape[1]

    row = edge_index[0]
    col = edge_index[1]

    s_p, sl_p, dis = _sc_edges(row, col)

    out = pl.pallas_call(
        _tc_final_body,
        out_shape=jax.ShapeDtypeStruct((1, out_dim), jnp.float32),
    )(x, s_p, sl_p, dis.reshape(1, N_PAD), W0, W1, b.reshape(1, out_dim))
    return out


# speedup vs baseline: 440.5539x; 1.1148x over previous
import functools

import jax
import jax.numpy as jnp
from jax import lax
from jax.experimental import pallas as pl
from jax.experimental.pallas import tpu as pltpu
from jax.experimental.pallas import tpu_sc as plsc

N_NODES = 10000
N_EDGES = 640000
N_PAD = 10240
NC = 2
NS = 16
NW = NC * NS
CHUNK = N_EDGES // NW
DCHUNK = N_EDGES // NS
L = 16
SLICE = N_PAD // NS
UNROLL_A = 10
UNROLL_C = 10

_mesh = plsc.VectorSubcoreMesh(core_axis_name="c", subcore_axis_name="s")
_sc_params = pltpu.CompilerParams(needs_layout_passes=False)


def _newton_rsqrt(x):
    i = plsc.bitcast(x, jnp.int32)
    i = 0x5F3759DF - lax.shift_right_logical(i, 1)
    y = plsc.bitcast(i, jnp.float32)
    half = x * -0.5
    for _ in range(3):
        y = y * (y * y * half + 1.5)
    return y


@functools.partial(
    pl.kernel,
    out_type=jax.ShapeDtypeStruct((NC, N_PAD), jnp.float32),
    mesh=_mesh,
    scratch_types=[
        pltpu.VMEM((DCHUNK,), jnp.int32),
        pltpu.VMEM((CHUNK,), jnp.int32),
        pltpu.VMEM((N_PAD,), jnp.float32),
        pltpu.VMEM((N_PAD,), jnp.float32),
        pltpu.VMEM((N_PAD,), jnp.float32),
        pltpu.VMEM((NS, SLICE), jnp.float32),
        pltpu.VMEM((NS, SLICE), jnp.float32),
        pltpu.VMEM_SHARED((NS, N_PAD), jnp.float32),
        pltpu.VMEM_SHARED((N_PAD,), jnp.float32),
        pltpu.SemaphoreType.DMA,
        pltpu.SemaphoreType.DMA,
    ],
    compiler_params=_sc_params,
)
def _sc_edges(ei_hbm, c_hbm, row_v, col_v, deg_v, s_v, sl_v, part_v, part2_v,
              degsh, dissh, sem, sem2):
    cid = lax.axis_index("c")
    sid = lax.axis_index("s")
    wid = sid * NC + cid
    cp_r = pltpu.async_copy(ei_hbm.at[pl.ds(sid * DCHUNK, DCHUNK)], row_v, sem)
    cp_c = pltpu.async_copy(
        ei_hbm.at[pl.ds(N_EDGES + wid * CHUNK, CHUNK)], col_v, sem2)

    zeros = jnp.zeros((L,), jnp.float32)

    def zero_body(i, carry):
        for u in range(8):
            deg_v[pl.ds((i * 8 + u) * L, L)] = zeros
            s_v[pl.ds((i * 8 + u) * L, L)] = zeros
            sl_v[pl.ds((i * 8 + u) * L, L)] = zeros
        return carry

    lax.fori_loop(0, N_PAD // L // 8, zero_body, 0)
    cp_r.wait()

    ones = jnp.ones((L,), jnp.float32)

    def deg_body(i, carry):
        rs = [row_v[pl.ds((i * UNROLL_A + u) * L, L)] for u in range(UNROLL_A)]
        for u in range(UNROLL_A):
            plsc.addupdate_scatter(deg_v, [rs[u]], ones)
        return carry

    lax.fori_loop(0, DCHUNK // L // UNROLL_A, deg_body, 0)

    pltpu.sync_copy(deg_v, degsh.at[sid])
    plsc.subcore_barrier()
    col0 = sid * SLICE
    pltpu.sync_copy(degsh.at[:, pl.ds(col0, SLICE)], part_v)

    def red_body(j, carry):
        acc = part_v[0, pl.ds(j * L, L)]
        for t in range(1, NS):
            acc = acc + part_v[t, pl.ds(j * L, L)]
        dis = _newton_rsqrt(acc)
        dis = jnp.where(acc > 0.0, dis, 0.0)
        deg_v[pl.ds(col0 + j * L, L)] = dis
        return carry

    lax.fori_loop(0, SLICE // L, red_body, 0)

    pltpu.sync_copy(deg_v.at[pl.ds(col0, SLICE)], dissh.at[pl.ds(col0, SLICE)])
    plsc.subcore_barrier()
    pltpu.sync_copy(dissh, deg_v)
    cp_c.wait()

    cbase = cid * CHUNK

    def s_body(i, carry):
        rs = []
        cs = []
        gs = []
        for u in range(UNROLL_C):
            off = (i * UNROLL_C + u) * L
            rs.append(row_v[pl.ds(cbase + off, L)])
            cs.append(col_v[pl.ds(off, L)])
            gs.append(plsc.load_gather(deg_v, [cs[u]]))
        for u in range(UNROLL_C):
            plsc.addupdate_scatter(s_v, [rs[u]], gs[u])
            plsc.addupdate_scatter(sl_v, [rs[u]], ones, mask=rs[u] == cs[u])
        return carry

    lax.fori_loop(0, CHUNK // L // UNROLL_C, s_body, 0)

    pltpu.sync_copy(s_v, degsh.at[sid])
    plsc.subcore_barrier()
    pltpu.sync_copy(degsh.at[:, pl.ds(col0, SLICE)], part_v)
    plsc.subcore_barrier()
    pltpu.sync_copy(sl_v, degsh.at[sid])
    plsc.subcore_barrier()
    pltpu.sync_copy(degsh.at[:, pl.ds(col0, SLICE)], part2_v)

    def c_body(j, carry):
        s_acc = part_v[0, pl.ds(j * L, L)]
        sl_acc = part2_v[0, pl.ds(j * L, L)]
        for t in range(1, NS):
            s_acc = s_acc + part_v[t, pl.ds(j * L, L)]
            sl_acc = sl_acc + part2_v[t, pl.ds(j * L, L)]
        dis = deg_v[pl.ds(col0 + j * L, L)]
        s_v[pl.ds(col0 + j * L, L)] = -(dis * s_acc) - sl_acc
        return carry

    lax.fori_loop(0, SLICE // L, c_body, 0)

    pltpu.sync_copy(s_v.at[pl.ds(col0, SLICE)],
                    c_hbm.at[cid, pl.ds(col0, SLICE)])


def _tc_final_body(x_ref, cp_ref, w0_ref, w1_ref, b_ref, out_ref):
    c = jnp.sum(cp_ref[...], axis=0, keepdims=True)
    ones = jnp.ones((1, N_NODES), jnp.float32)
    cm = jnp.concatenate([ones, c[:, :N_NODES]], axis=0)
    x = x_ref[...]
    v = jnp.dot(cm, x, preferred_element_type=jnp.float32)
    out = (
        jnp.dot(v[0:1], w0_ref[...], preferred_element_type=jnp.float32)
        + jnp.dot(v[1:2], w1_ref[...], preferred_element_type=jnp.float32)
    ) * (1.0 / N_NODES) + b_ref[...]
    out_ref[...] = out


@jax.jit
def kernel(x, edge_index, W0, W1, b):
    out_dim = W0.shape[1]

    ei_flat = edge_index.reshape(2 * N_EDGES)
    c_p = _sc_edges(ei_flat)

    out = pl.pallas_call(
        _tc_final_body,
        out_shape=jax.ShapeDtypeStruct((1, out_dim), jnp.float32),
    )(x, c_p, W0, W1, b.reshape(1, out_dim))
    return out
